# Initial kernel scaffold; baseline (speedup 1.0000x reference)
import jax
import jax.numpy as jnp
from jax.experimental import pallas as pl


def kernel(partial_xyz, partial_features, global_feature, params):
    raise NotImplementedError("write your pallas kernel here")


# speedup vs baseline: 12.1994x; 12.1994x over previous
import functools

import jax
import jax.numpy as jnp
from jax import lax
from jax.experimental import pallas as pl
from jax.experimental.pallas import tpu as pltpu
from jax.experimental.pallas import tpu_sc as plsc

IN_DIM = 128
OUT_DIM = 128
HID = 64
GDIM = 512
KNN = 20
UP = 2
NB = 4
NPTS = 2048
BNTOT = NB * NPTS
TBL_W = 128

RA = 512
BQ = 256
BN = 128
KBN = KNN * BN
NBLK = NPTS // BN


def _proj_body(pf_ref, w1_ref, b1_ref, w2_ref, b2_ref, wsc_ref, bsc_ref,
               wq_ref, bq_ref, wk_ref, bk_ref, wv_ref, bv_ref,
               q_ref, val_ref, tab_ref):
    pf = pf_ref[...]
    h = jnp.maximum(pf @ w1_ref[...] + b1_ref[...], 0.0)
    val = h @ w2_ref[...] + b2_ref[...] + pf @ wsc_ref[...] + bsc_ref[...]
    q_ref[...] = pf @ wq_ref[...] + bq_ref[...]
    val_ref[...] = val
    kf = pf @ wk_ref[...] + bk_ref[...]
    v = val @ wv_ref[...] + bv_ref[...]
    tab_ref[...] = jnp.concatenate([kf, v], axis=1)


def _run_proj(pf, p):
    w1 = p['mlpv_W1'][:IN_DIM] + p['mlpv_W1'][IN_DIM:]
    wsc = p['mlpv_Wsc'][:IN_DIM] + p['mlpv_Wsc'][IN_DIM:]
    row = lambda b: b.reshape(1, -1)
    weights = [w1, row(p['mlpv_b1']), p['mlpv_W2'], row(p['mlpv_b2']),
               wsc, row(p['mlpv_bsc']),
               p['Wq'], row(p['bq']), p['Wk'], row(p['bk']), p['Wv'], row(p['bv'])]
    wspecs = [pl.BlockSpec(w.shape, lambda i: (0, 0)) for w in weights]
    return pl.pallas_call(
        _proj_body,
        grid=(BNTOT // RA,),
        in_specs=[pl.BlockSpec((RA, IN_DIM), lambda i: (i, 0))] + wspecs,
        out_specs=[pl.BlockSpec((RA, HID), lambda i: (i, 0)),
                   pl.BlockSpec((RA, OUT_DIM), lambda i: (i, 0)),
                   pl.BlockSpec((RA, TBL_W), lambda i: (i, 0))],
        out_shape=[jax.ShapeDtypeStruct((BNTOT, HID), jnp.float32),
                   jax.ShapeDtypeStruct((BNTOT, OUT_DIM), jnp.float32),
                   jax.ShapeDtypeStruct((BNTOT, TBL_W), jnp.float32)],
    )(pf, *weights)


def _knn_body(posq_ref, posT_ref, idx_ref):
    pq = posq_ref[0]
    kT = posT_ref[0]
    d = ((pq[:, 0:1] - kT[0:1, :]) ** 2
         + (pq[:, 1:2] - kT[1:2, :]) ** 2
         + (pq[:, 2:3] - kT[2:3, :]) ** 2)
    col = lax.broadcasted_iota(jnp.int32, (BQ, NPTS), 1)
    colk = lax.broadcasted_iota(jnp.int32, (BQ, KNN), 1)
    idxm = jnp.zeros((BQ, KNN), jnp.int32)
    for kk in range(KNN):
        m = jnp.min(d, axis=1, keepdims=True)
        cand = jnp.where(d == m, col, NPTS)
        j = jnp.min(cand, axis=1, keepdims=True)
        idxm = jnp.where(colk == kk, j, idxm)
        d = jnp.where(col == j, jnp.inf, d)
    idx_ref[0] = idxm + pl.program_id(0) * NPTS


def _run_knn(pos, posT):
    return pl.pallas_call(
        _knn_body,
        grid=(NB, NPTS // BQ),
        in_specs=[pl.BlockSpec((1, BQ, 3), lambda b, i: (b, i, 0)),
                  pl.BlockSpec((1, 3, NPTS), lambda b, i: (b, 0, 0))],
        out_specs=pl.BlockSpec((1, BQ, KNN), lambda b, i: (b, i, 0)),
        out_shape=jax.ShapeDtypeStruct((NB, NPTS, KNN), jnp.int32),
    )(pos, posT)


_NC = 2
_NS = 16
_NW = _NC * _NS
_GCH = 128


@functools.lru_cache(maxsize=None)
def _make_sc_gather(n_idx, width, n_pts):
    per_w = n_idx // _NW
    n_ch = per_w // _GCH
    mesh = plsc.VectorSubcoreMesh(core_axis_name="c", subcore_axis_name="s",
                                  num_cores=_NC, num_subcores=_NS)

    @functools.partial(
        pl.kernel, mesh=mesh,
        compiler_params=pltpu.CompilerParams(needs_layout_passes=False),
        out_type=[jax.ShapeDtypeStruct((n_idx, width), jnp.float32),
                  jax.ShapeDtypeStruct((n_idx * 8,), jnp.float32)],
        scratch_types=[pltpu.VMEM((_GCH,), jnp.int32),
                       pltpu.VMEM((_GCH, width), jnp.float32),
                       pltpu.VMEM((_GCH * 8,), jnp.float32),
                       pltpu.VMEM((n_pts,), jnp.float32),
                       pltpu.VMEM((n_pts,), jnp.float32),
                       pltpu.VMEM((n_pts,), jnp.float32),
                       pltpu.SemaphoreType.DMA],
    )
    def gather_k(tab_hbm, x_hbm, y_hbm, z_hbm, idx_hbm, kv_hbm, pos8_hbm,
                 idx_v, rows_v, pos8_v, xv, yv, zv, sem):
        wid = lax.axis_index("s") * _NC + lax.axis_index("c")
        base = wid * per_w
        pltpu.sync_copy(x_hbm, xv)
        pltpu.sync_copy(y_hbm, yv)
        pltpu.sync_copy(z_hbm, zv)
        zero16 = jnp.zeros((16,), jnp.float32)

        def zbody(i, carry):
            pos8_v[pl.ds(i * 16, 16)] = zero16
            return carry

        lax.fori_loop(0, _GCH * 8 // 16, zbody, 0)
        lanes = lax.iota(jnp.int32, 16)

        def body(c, carry):
            off = base + c * _GCH
            pltpu.sync_copy(idx_hbm.at[pl.ds(off, _GCH)], idx_v)
            cp = pltpu.async_copy(tab_hbm.at[idx_v], rows_v, sem)
            for c2 in range(_GCH // 16):
                ivec = idx_v[pl.ds(c2 * 16, 16)]
                rows = (lanes + c2 * 16) * 8
                plsc.store_scatter(pos8_v, [rows], plsc.load_gather(xv, [ivec]))
                plsc.store_scatter(pos8_v, [rows + 1], plsc.load_gather(yv, [ivec]))
                plsc.store_scatter(pos8_v, [rows + 2], plsc.load_gather(zv, [ivec]))
            pltpu.sync_copy(pos8_v, pos8_hbm.at[pl.ds(off * 8, _GCH * 8)])
            cp.wait()
            pltpu.sync_copy(rows_v, kv_hbm.at[pl.ds(off, _GCH)])
            return carry

        lax.fori_loop(0, n_ch, body, 0)

    return gather_k


def _sc_gather(tab, pos3, idx_flat):
    kv, p8 = _make_sc_gather(BNTOT * KNN, TBL_W, BNTOT)(
        tab, pos3[0], pos3[1], pos3[2], idx_flat)
    return kv, p8.reshape(-1, 8)


def _attn_body(G_ref, pg_ref, q_ref, posp_ref, val_ref, g_ref,
               posW1, posb1, posW2, posb2, aW1, ab1, WtR, btR, endW, endb,
               m1W1a, m1W1b, m1b1, m1W2, m1b2, m1Wsca, m1Wscb, m1bsc,
               m2W1, m2b1, m2W2, m2b2, m2Wsc, m2bsc,
               m3W1a, m3W1b, m3b1, m3W2, m3b2, m3Wsca, m3Wscb, m3bsc,
               m4W1, m4b1, m4W2, m4b2,
               sf_ref, xyz_ref):
    Gb = G_ref[...]
    kf = Gb[:, :HID]
    vg = Gb[:, HID:]
    pg8 = pg_ref[...]
    qb = q_ref[...]
    pq8 = posp_ref[...]
    qrep = jnp.concatenate([qb] * KNN, axis=0)
    prep = jnp.concatenate([pq8] * KNN, axis=0)
    pr8 = prep - pg8
    pe = jnp.maximum(pr8 @ posW1[...] + posb1[...], 0.0) @ posW2[...] + posb2[...]
    h = jnp.maximum((qrep - kf + pe) @ aW1[...] + ab1[...], 0.0)
    attn = h @ WtR[...] + btR[...]
    w = vg + pe
    prod = attn * jnp.concatenate([w, w], axis=1)
    agg = prod[0:BN]
    for kk in range(1, KNN):
        agg = agg + prod[kk * BN:(kk + 1) * BN]
    val = val_ref[...]
    y0 = agg[:, :HID] @ endW[...] + endb[...] + val
    y1 = agg[:, HID:] @ endW[...] + endb[...] + val
    f = jnp.concatenate([y0, y1], axis=0)
    g = g_ref[0]
    h1 = jnp.maximum(f @ m1W1a[...] + g @ m1W1b[...] + m1b1[...], 0.0)
    f1 = h1 @ m1W2[...] + m1b2[...] + f @ m1Wsca[...] + g @ m1Wscb[...] + m1bsc[...]
    h2 = jnp.maximum(f1 @ m2W1[...] + m2b1[...], 0.0)
    f2 = h2 @ m2W2[...] + m2b2[...] + f1 @ m2Wsc[...] + m2bsc[...]
    h3 = jnp.maximum(f2 @ m3W1a[...] + g @ m3W1b[...] + m3b1[...], 0.0)
    sf = h3 @ m3W2[...] + m3b2[...] + f2 @ m3Wsca[...] + g @ m3Wscb[...] + m3bsc[...]
    h4 = jnp.maximum(sf @ m4W1[...] + m4b1[...], 0.0)
    xyz = jnp.tanh(h4 @ m4W2[...] + m4b2[...])
    sf_ref[...] = sf
    xyz_ref[...] = xyz


def _run_attn(G, pg8, qf, posp, valf, gf2, p):
    row = lambda b: b.reshape(1, -1)
    posW1p = jnp.pad(p['pos_W1'], ((0, 5), (0, 0)))
    WtR = p['attn_Wt'].transpose(0, 2, 1).reshape(4 * HID, 2 * HID)
    btR = jnp.tile(p['attn_bt'], UP).reshape(1, -1)
    weights = [posW1p, row(p['pos_b1']), p['pos_W2'], row(p['pos_b2']),
               p['attn_W1'], row(p['attn_b1']), WtR, btR,
               p['end_W'], row(p['end_b']),
               p['m1_W1'][:OUT_DIM], p['m1_W1'][OUT_DIM:], row(p['m1_b1']),
               p['m1_W2'], row(p['m1_b2']),
               p['m1_Wsc'][:OUT_DIM], p['m1_Wsc'][OUT_DIM:], row(p['m1_bsc']),
               p['m2_W1'], row(p['m2_b1']), p['m2_W2'], row(p['m2_b2']),
               p['m2_Wsc'], row(p['m2_bsc']),
               p['m3_W1'][:OUT_DIM], p['m3_W1'][OUT_DIM:], row(p['m3_b1']),
               p['m3_W2'], row(p['m3_b2']),
               p['m3_Wsc'][:OUT_DIM], p['m3_Wsc'][OUT_DIM:], row(p['m3_bsc']),
               p['m4_W1'], row(p['m4_b1']), p['m4_W2'], row(p['m4_b2'])]
    wspecs = [pl.BlockSpec(w.shape, lambda t: (0, 0)) for w in weights]
    nt = NB * NBLK
    return pl.pallas_call(
        _attn_body,
        grid=(nt,),
        in_specs=[pl.BlockSpec((KBN, TBL_W), lambda t: (t, 0)),
                  pl.BlockSpec((KBN, 8), lambda t: (t, 0)),
                  pl.BlockSpec((BN, HID), lambda t: (t, 0)),
                  pl.BlockSpec((BN, 8), lambda t: (t, 0)),
                  pl.BlockSpec((BN, OUT_DIM), lambda t: (t, 0)),
                  pl.BlockSpec((1, 1, GDIM), lambda t: (t // NBLK, 0, 0))] + wspecs,
        out_specs=[pl.BlockSpec((UP * BN, OUT_DIM), lambda t: (t, 0)),
                   pl.BlockSpec((UP * BN, 3), lambda t: (t, 0))],
        out_shape=[jax.ShapeDtypeStruct((nt * UP * BN, OUT_DIM), jnp.float32),
                   jax.ShapeDtypeStruct((nt * UP * BN, 3), jnp.float32)],
    )(G, pg8, qf, posp, valf, gf2, *weights)


@jax.jit
def kernel(partial_xyz, partial_features, global_feature, params):
    pf = partial_features.reshape(BNTOT, IN_DIM)
    posp = jnp.pad(partial_xyz.reshape(BNTOT, 3), ((0, 0), (0, 5)))
    posT = jnp.transpose(partial_xyz, (0, 2, 1))
    pos3 = partial_xyz.transpose(2, 0, 1).reshape(3, BNTOT)

    qf, valf, tab = _run_proj(pf, params)
    idx = _run_knn(partial_xyz, posT)

    idx_flat = (idx.reshape(NB, NBLK, BN, KNN)
                   .transpose(0, 1, 3, 2)
                   .reshape(BNTOT * KNN))
    G, pg8 = _sc_gather(tab, pos3, idx_flat)

    gf2 = global_feature.reshape(NB, 1, GDIM)
    sf, xyz = _run_attn(G, pg8, qf, posp, valf, gf2, params)

    seed_features = (sf.reshape(NB, NBLK, UP, BN, OUT_DIM)
                       .transpose(0, 1, 3, 2, 4)
                       .reshape(NB, NPTS * UP, OUT_DIM))
    seed_xyz = (xyz.reshape(NB, NBLK, UP, BN, 3)
                   .transpose(0, 1, 3, 2, 4)
                   .reshape(NB, NPTS * UP, 3))
    return seed_xyz, seed_features
